# baseline (speedup 1.0000x reference)
import functools

import jax
import jax.numpy as jnp
import numpy as np
from jax.experimental import pallas as pl
from jax.experimental.pallas import tpu as pltpu

S = 2048
D_MODEL = 1024
NUM_HEADS = 16
NUM_KV_HEADS = 4
HEAD_DIM = 64
N_REP = NUM_HEADS // NUM_KV_HEADS
KV_D = NUM_KV_HEADS * HEAD_DIM
_PID = (0, 1, 2, 3, 50, 100)
_LAYER_IDX = 8
_KK = int(192 - _LAYER_IDX / 31 * (192 - 64))

_HI = jax.lax.Precision.HIGHEST
_ZROWS = 1024


def _mega_kernel(
    x_ref, wq_ref, wkv_ref, wo_ref,
    pruned_ref, out_ref, kv_ref,
    zbuf, prow_buf, olast_buf, sems,
):
    zbuf[...] = jnp.zeros_like(zbuf)
    copies = []
    n = 0
    nblk = (S - 8) // _ZROWS
    rem = (S - 8) - nblk * _ZROWS
    for h in range(NUM_HEADS):
        for j in range(nblk):
            c = pltpu.make_async_copy(
                zbuf,
                pruned_ref.at[h, j * _ZROWS : (j + 1) * _ZROWS, :],
                sems.at[n],
            )
            c.start()
            copies.append(c)
            n += 1
        c = pltpu.make_async_copy(
            zbuf.at[0:rem, :],
            pruned_ref.at[h, nblk * _ZROWS : S - 8, :],
            sems.at[n],
        )
        c.start()
        copies.append(c)
        n += 1
    for j in range(nblk):
        c = pltpu.make_async_copy(
            zbuf.at[:, 0:D_MODEL],
            out_ref.at[j * _ZROWS : (j + 1) * _ZROWS, :],
            sems.at[n],
        )
        c.start()
        copies.append(c)
        n += 1
    c = pltpu.make_async_copy(
        zbuf.at[0:rem, 0:D_MODEL],
        out_ref.at[nblk * _ZROWS : S - 8, :],
        sems.at[n],
    )
    c.start()
    copies.append(c)
    n += 1

    half = S // 2
    for i in range(2):
        kv_ref[i * half : (i + 1) * half, :] = jnp.dot(
            x_ref[i * half : (i + 1) * half, :].astype(jnp.bfloat16),
            wkv_ref[...].astype(jnp.bfloat16),
            preferred_element_type=jnp.float32,
        )
    k = kv_ref[:, :KV_D]
    v = kv_ref[:, KV_D:]

    q = jnp.dot(
        x_ref[S - 1 : S, :].astype(jnp.bfloat16),
        wq_ref[...].astype(jnp.bfloat16),
        preferred_element_type=jnp.float32,
    )
    rows = []
    for h in range(NUM_HEADS):
        qh = q[:, h * HEAD_DIM : (h + 1) * HEAD_DIM].astype(jnp.bfloat16)
        g = h // N_REP
        kg = k[:, g * HEAD_DIM : (g + 1) * HEAD_DIM].astype(jnp.bfloat16)
        rows.append(
            jax.lax.dot_general(
                qh, kg, (((1,), (1,)), ((), ())),
                preferred_element_type=jnp.float32,
            )
        )
    scores = jnp.concatenate(rows, axis=0) * (
        1.0 / np.sqrt(HEAD_DIM)
    )
    m = jnp.max(scores, axis=1, keepdims=True)
    e = jnp.exp(scores - m)
    attn = e / jnp.sum(e, axis=1, keepdims=True)

    imp = jnp.mean(attn, axis=0, keepdims=True)
    lane = jax.lax.broadcasted_iota(jnp.int32, (1, S), 1)
    is_pid = functools.reduce(jnp.logical_or, [lane == p for p in _PID])
    imp = jnp.where(is_pid, imp * 2.5, imp)

    bits = jax.lax.bitcast_convert_type(imp, jnp.int32)

    def body(i, t):
        cand = t | jax.lax.shift_left(jnp.int32(1), 30 - i)
        cnt = jnp.sum((bits >= cand).astype(jnp.int32))
        return jnp.where(cnt >= _KK, cand, t)

    t = jax.lax.fori_loop(0, 31, body, jnp.int32(0))

    gt = bits > t
    eq = bits == t
    need = (_KK - jnp.sum(gt.astype(jnp.int32))).astype(jnp.float32)
    c32 = eq.astype(jnp.float32)
    sh = 1
    while sh < S:
        c32 = c32 + jnp.concatenate(
            [jnp.zeros((1, sh), c32.dtype), c32[:, : S - sh]], axis=1
        )
        sh *= 2
    sel = jnp.logical_or(gt, jnp.logical_and(eq, c32 <= need))

    prow = attn * sel.astype(jnp.float32)
    prow_buf[...] = jnp.zeros_like(prow_buf)
    for h in range(NUM_HEADS):
        prow_buf[8 * h + 7 : 8 * h + 8, :] = prow[h : h + 1, :]

    ctx = jnp.dot(
        prow.astype(jnp.bfloat16),
        v.astype(jnp.bfloat16),
        preferred_element_type=jnp.float32,
    )
    hh = jax.lax.broadcasted_iota(jnp.int32, (NUM_HEADS, KV_D), 0)
    gg = jax.lax.broadcasted_iota(jnp.int32, (NUM_HEADS, KV_D), 1) // HEAD_DIM
    ctx = jnp.where(hh // N_REP == gg, ctx, 0.0)
    ctx16 = (
        ctx[:, 0:64] + ctx[:, 64:128] + ctx[:, 128:192] + ctx[:, 192:256]
    )

    olast = jnp.zeros((1, D_MODEL), jnp.float32)
    for h in range(NUM_HEADS):
        olast = olast + jnp.dot(
            ctx16[h : h + 1, :].astype(jnp.bfloat16),
            wo_ref[h * HEAD_DIM : (h + 1) * HEAD_DIM, :].astype(jnp.bfloat16),
            preferred_element_type=jnp.float32,
        )
    olast_buf[...] = jnp.zeros_like(olast_buf)
    olast_buf[7:8, :] = olast

    for h in range(NUM_HEADS):
        c = pltpu.make_async_copy(
            prow_buf.at[8 * h : 8 * (h + 1), :],
            pruned_ref.at[h, S - 8 : S, :],
            sems.at[n],
        )
        c.start()
        copies.append(c)
        n += 1
    c = pltpu.make_async_copy(olast_buf, out_ref.at[S - 8 : S, :], sems.at[n])
    c.start()
    copies.append(c)
    n += 1

    for c in copies:
        c.wait()


def kernel(hidden_states, Wq, Wk, Wv, Wo):
    x = hidden_states[0]
    Wkv = jnp.concatenate([Wk, Wv], axis=1)

    nsem = 2 * NUM_HEADS + 2 + NUM_HEADS + 1 + 1
    pruned, out, kv = pl.pallas_call(
        _mega_kernel,
        in_specs=[
            pl.BlockSpec(memory_space=pltpu.MemorySpace.VMEM),
            pl.BlockSpec(memory_space=pltpu.MemorySpace.VMEM),
            pl.BlockSpec(memory_space=pltpu.MemorySpace.VMEM),
            pl.BlockSpec(memory_space=pltpu.MemorySpace.VMEM),
        ],
        out_specs=(
            pl.BlockSpec(memory_space=pltpu.MemorySpace.HBM),
            pl.BlockSpec(memory_space=pltpu.MemorySpace.HBM),
            pl.BlockSpec(memory_space=pltpu.MemorySpace.VMEM),
        ),
        out_shape=(
            jax.ShapeDtypeStruct((NUM_HEADS, S, S), jnp.float32),
            jax.ShapeDtypeStruct((S, D_MODEL), jnp.float32),
            jax.ShapeDtypeStruct((S, 2 * KV_D), jnp.float32),
        ),
        scratch_shapes=[
            pltpu.VMEM((_ZROWS, S), jnp.float32),
            pltpu.VMEM((8 * NUM_HEADS, S), jnp.float32),
            pltpu.VMEM((8, D_MODEL), jnp.float32),
            pltpu.SemaphoreType.DMA((128,)),
        ],
    )(x, Wq, Wkv, Wo)

    k_flat = kv[:, :KV_D]
    v_flat = kv[:, KV_D:]
    k_kv = k_flat.reshape(1, S, NUM_KV_HEADS, HEAD_DIM).transpose(0, 2, 1, 3)
    v_kv = v_flat.reshape(1, S, NUM_KV_HEADS, HEAD_DIM).transpose(0, 2, 1, 3)
    return out[None], pruned[None], k_kv, v_kv


# speedup vs baseline: 1.1009x; 1.1009x over previous
import functools

import jax
import jax.numpy as jnp
import numpy as np
from jax.experimental import pallas as pl
from jax.experimental.pallas import tpu as pltpu

S = 2048
D_MODEL = 1024
NUM_HEADS = 16
NUM_KV_HEADS = 4
HEAD_DIM = 64
N_REP = NUM_HEADS // NUM_KV_HEADS
KV_D = NUM_KV_HEADS * HEAD_DIM
_PID = (0, 1, 2, 3, 50, 100)
_LAYER_IDX = 8
_KK = int(192 - _LAYER_IDX / 31 * (192 - 64))

_ZROWS = 512


def _bf16(a):
    return a.astype(jnp.bfloat16)


def _mega_kernel(
    x_hbm, wq_hbm, wkv_hbm, wo_hbm,
    pruned_ref, out_ref, kv_hbm,
    xbuf, wqbuf, wkvbuf, wobuf, kvbuf, zbuf, prow_buf, olast_buf, sems,
):
    in_x = pltpu.make_async_copy(x_hbm, xbuf, sems.at[120])
    in_q = pltpu.make_async_copy(wq_hbm, wqbuf, sems.at[121])
    in_kv = pltpu.make_async_copy(wkv_hbm, wkvbuf, sems.at[122])
    in_o = pltpu.make_async_copy(wo_hbm, wobuf, sems.at[123])
    for c in (in_x, in_kv, in_q, in_o):
        c.start()

    zbuf[...] = jnp.zeros_like(zbuf)
    copies = []
    n = 0
    nblk = (S - 8) // _ZROWS
    rem = (S - 8) - nblk * _ZROWS
    for h in range(NUM_HEADS):
        for j in range(nblk):
            c = pltpu.make_async_copy(
                zbuf,
                pruned_ref.at[h, j * _ZROWS : (j + 1) * _ZROWS, :],
                sems.at[n],
            )
            c.start()
            copies.append(c)
            n += 1
        c = pltpu.make_async_copy(
            zbuf.at[0:rem, :],
            pruned_ref.at[h, nblk * _ZROWS : S - 8, :],
            sems.at[n],
        )
        c.start()
        copies.append(c)
        n += 1
    for j in range(nblk):
        c = pltpu.make_async_copy(
            zbuf.at[:, 0:D_MODEL],
            out_ref.at[j * _ZROWS : (j + 1) * _ZROWS, :],
            sems.at[n],
        )
        c.start()
        copies.append(c)
        n += 1
    c = pltpu.make_async_copy(
        zbuf.at[0:rem, 0:D_MODEL],
        out_ref.at[nblk * _ZROWS : S - 8, :],
        sems.at[n],
    )
    c.start()
    copies.append(c)
    n += 1

    in_x.wait()
    in_kv.wait()
    half = S // 2
    for i in range(2):
        kvbuf[i * half : (i + 1) * half, :] = jnp.dot(
            _bf16(xbuf[i * half : (i + 1) * half, :]),
            _bf16(wkvbuf[...]),
            preferred_element_type=jnp.float32,
        )
        c = pltpu.make_async_copy(
            kvbuf.at[i * half : (i + 1) * half, :],
            kv_hbm.at[i * half : (i + 1) * half, :],
            sems.at[n],
        )
        c.start()
        copies.append(c)
        n += 1
    k = kvbuf[:, :KV_D]
    v = kvbuf[:, KV_D:]

    in_q.wait()
    q = jnp.dot(
        _bf16(xbuf[S - 1 : S, :]),
        _bf16(wqbuf[...]),
        preferred_element_type=jnp.float32,
    )
    rows = []
    for h in range(NUM_HEADS):
        qh = _bf16(q[:, h * HEAD_DIM : (h + 1) * HEAD_DIM])
        g = h // N_REP
        kg = _bf16(k[:, g * HEAD_DIM : (g + 1) * HEAD_DIM])
        rows.append(
            jax.lax.dot_general(
                qh, kg, (((1,), (1,)), ((), ())),
                preferred_element_type=jnp.float32,
            )
        )
    scores = jnp.concatenate(rows, axis=0) * (
        1.0 / np.sqrt(HEAD_DIM)
    )
    m = jnp.max(scores, axis=1, keepdims=True)
    e = jnp.exp(scores - m)
    attn = e / jnp.sum(e, axis=1, keepdims=True)

    imp = jnp.mean(attn, axis=0, keepdims=True)
    lane = jax.lax.broadcasted_iota(jnp.int32, (1, S), 1)
    is_pid = functools.reduce(jnp.logical_or, [lane == p for p in _PID])
    imp = jnp.where(is_pid, imp * 2.5, imp)

    bits = jax.lax.bitcast_convert_type(imp, jnp.int32)

    def body(i, t):
        cand = t | jax.lax.shift_left(jnp.int32(1), 30 - i)
        cnt = jnp.sum((bits >= cand).astype(jnp.int32))
        return jnp.where(cnt >= _KK, cand, t)

    t = jax.lax.fori_loop(0, 31, body, jnp.int32(0))

    gt = bits > t
    eq = bits == t
    need = (_KK - jnp.sum(gt.astype(jnp.int32))).astype(jnp.float32)
    c32 = eq.astype(jnp.float32)
    sh = 1
    while sh < S:
        c32 = c32 + jnp.concatenate(
            [jnp.zeros((1, sh), c32.dtype), c32[:, : S - sh]], axis=1
        )
        sh *= 2
    sel = jnp.logical_or(gt, jnp.logical_and(eq, c32 <= need))

    prow = attn * sel.astype(jnp.float32)
    prow_buf[...] = jnp.zeros_like(prow_buf)
    for h in range(NUM_HEADS):
        prow_buf[8 * h + 7 : 8 * h + 8, :] = prow[h : h + 1, :]

    ctx = jnp.dot(
        _bf16(prow), _bf16(v), preferred_element_type=jnp.float32
    )
    hh = jax.lax.broadcasted_iota(jnp.int32, (NUM_HEADS, KV_D), 0)
    gg = jax.lax.broadcasted_iota(jnp.int32, (NUM_HEADS, KV_D), 1) // HEAD_DIM
    ctx = jnp.where(hh // N_REP == gg, ctx, 0.0)
    ctx16 = (
        ctx[:, 0:64] + ctx[:, 64:128] + ctx[:, 128:192] + ctx[:, 192:256]
    )

    in_o.wait()
    olast = jnp.zeros((1, D_MODEL), jnp.float32)
    for h in range(NUM_HEADS):
        olast = olast + jnp.dot(
            _bf16(ctx16[h : h + 1, :]),
            _bf16(wobuf[h * HEAD_DIM : (h + 1) * HEAD_DIM, :]),
            preferred_element_type=jnp.float32,
        )
    olast_buf[...] = jnp.zeros_like(olast_buf)
    olast_buf[7:8, :] = olast

    for h in range(NUM_HEADS):
        c = pltpu.make_async_copy(
            prow_buf.at[8 * h : 8 * (h + 1), :],
            pruned_ref.at[h, S - 8 : S, :],
            sems.at[n],
        )
        c.start()
        copies.append(c)
        n += 1
    c = pltpu.make_async_copy(olast_buf, out_ref.at[S - 8 : S, :], sems.at[n])
    c.start()
    copies.append(c)
    n += 1

    for c in copies:
        c.wait()


def kernel(hidden_states, Wq, Wk, Wv, Wo):
    x = hidden_states[0]
    Wkv = jnp.concatenate([Wk, Wv], axis=1)

    hbm = pl.BlockSpec(memory_space=pltpu.MemorySpace.HBM)
    pruned, out, kv = pl.pallas_call(
        _mega_kernel,
        in_specs=[hbm, hbm, hbm, hbm],
        out_specs=(hbm, hbm, hbm),
        out_shape=(
            jax.ShapeDtypeStruct((NUM_HEADS, S, S), jnp.float32),
            jax.ShapeDtypeStruct((S, D_MODEL), jnp.float32),
            jax.ShapeDtypeStruct((S, 2 * KV_D), jnp.float32),
        ),
        scratch_shapes=[
            pltpu.VMEM((S, D_MODEL), jnp.float32),
            pltpu.VMEM((D_MODEL, D_MODEL), jnp.float32),
            pltpu.VMEM((D_MODEL, 2 * KV_D), jnp.float32),
            pltpu.VMEM((D_MODEL, D_MODEL), jnp.float32),
            pltpu.VMEM((S, 2 * KV_D), jnp.float32),
            pltpu.VMEM((_ZROWS, S), jnp.float32),
            pltpu.VMEM((8 * NUM_HEADS, S), jnp.float32),
            pltpu.VMEM((8, D_MODEL), jnp.float32),
            pltpu.SemaphoreType.DMA((128,)),
        ],
    )(x, Wq, Wkv, Wo)

    k_flat = kv[:, :KV_D]
    v_flat = kv[:, KV_D:]
    k_kv = k_flat.reshape(1, S, NUM_KV_HEADS, HEAD_DIM).transpose(0, 2, 1, 3)
    v_kv = v_flat.reshape(1, S, NUM_KV_HEADS, HEAD_DIM).transpose(0, 2, 1, 3)
    return out[None], pruned[None], k_kv, v_kv
